# baseline (speedup 1.0000x reference)
import jax
import jax.numpy as jnp
from jax import lax
from jax.experimental import pallas as pl
from jax.experimental.pallas import tpu as pltpu
from jax.experimental.pallas import tpu_sc as plsc

_NUM_CLASSES = 10
_MAX_STEPS = 10
_T = 8388608
_NUM_STEPS = int(min(_MAX_STEPS, _T // 2))

_BIDX = (2022204, 2302723, 6800451, 5485289, 6417134,
         1160667, 5296668, 341701, 1583860)
_CM = (5, 2, 4, 2, 5, 4, 1, 8, 6, 5)

_NC, _NS, _L = 2, 16, 16
_NW = _NC * _NS
_SC_N = 2621440
_S_OFF = _T - _SC_N
_PW = _SC_N // _NW
_CH = 20480
_NCHUNK = _PW // _CH
_NPAIR = _NCHUNK // 2

_BS = 524288
_TC_GRID = _S_OFF // _BS


def _sc_body(x_hbm, b_hbm, out_hbm,
             b_v, xb0, xb1, ob0, ob1, si0, si1, so0, so1):
    xbufs, obufs = (xb0, xb1), (ob0, ob1)
    sin, sout = (si0, si1), (so0, so1)
    wid = lax.axis_index("s") * _NC + lax.axis_index("c")
    base = _S_OFF + wid * _PW

    for b in range(2):
        pltpu.async_copy(x_hbm.at[pl.ds(base + b * _CH, _CH)], xbufs[b],
                         sin[b])

    pltpu.sync_copy(b_hbm, b_v)
    bvec = b_v[...]
    bs = [jnp.full((_L,), bvec[j], jnp.float32) for j in range(_NUM_STEPS - 1)]
    cms = [jnp.full((_L,), _CM[j], jnp.int32) for j in range(_NUM_STEPS)]

    def pair_body(p, _):
        for b in range(2):
            off = base + (2 * p + b) * _CH
            pltpu.make_async_copy(
                x_hbm.at[pl.ds(off, _CH)], xbufs[b], sin[b]).wait()

            @pl.when(p > 0)
            def _wait_out():
                pltpu.make_async_copy(
                    obufs[b], out_hbm.at[pl.ds(off - 2 * _CH - _S_OFF, _CH)],
                    sout[b]).wait()

            @plsc.parallel_loop(0, _CH, step=_L, unroll=8)
            def _vecs(i, b=b):
                x = xbufs[b][pl.ds(i, _L)]
                acc = cms[0]
                for j in range(_NUM_STEPS - 1):
                    acc = jnp.where(x > bs[j], cms[j + 1], acc)
                obufs[b][pl.ds(i, _L)] = acc

            pltpu.async_copy(obufs[b], out_hbm.at[pl.ds(off - _S_OFF, _CH)],
                             sout[b])

            @pl.when(p < _NPAIR - 1)
            def _issue_next():
                pltpu.async_copy(
                    x_hbm.at[pl.ds(off + 2 * _CH, _CH)], xbufs[b], sin[b])
        return 0

    lax.fori_loop(0, _NPAIR, pair_body, 0)
    for b in range(2):
        pltpu.make_async_copy(
            obufs[b],
            out_hbm.at[pl.ds(wid * _PW + (_NCHUNK - 2 + b) * _CH, _CH)],
            sout[b]).wait()


def _tc_body(b_s, x_ref, out_ref):
    x = x_ref[...]
    acc = jnp.full(x.shape, _CM[0], jnp.int32)
    for j in range(_NUM_STEPS - 1):
        acc = jnp.where(x > b_s[j], jnp.int32(_CM[j + 1]), acc)
    out_ref[...] = acc


def kernel(input):
    boundaries = jnp.sort(input[jnp.array(_BIDX, jnp.int32)])
    b16 = jnp.zeros((_L,), jnp.float32).at[: _NUM_STEPS - 1].set(boundaries)

    mesh = plsc.VectorSubcoreMesh(core_axis_name="c", subcore_axis_name="s")
    sc_run = pl.kernel(
        _sc_body,
        out_type=jax.ShapeDtypeStruct((_SC_N,), jnp.int32),
        mesh=mesh,
        scratch_types=[
            pltpu.VMEM((_L,), jnp.float32),
            pltpu.VMEM((_CH,), jnp.float32),
            pltpu.VMEM((_CH,), jnp.float32),
            pltpu.VMEM((_CH,), jnp.int32),
            pltpu.VMEM((_CH,), jnp.int32),
            pltpu.SemaphoreType.DMA,
            pltpu.SemaphoreType.DMA,
            pltpu.SemaphoreType.DMA,
            pltpu.SemaphoreType.DMA,
        ],
    )
    sc_out = sc_run(input, b16)

    tc_full = pl.pallas_call(
        _tc_body,
        grid=(_TC_GRID,),
        in_specs=[
            pl.BlockSpec(memory_space=pltpu.SMEM),
            pl.BlockSpec((_BS,), lambda i: (i,)),
        ],
        out_specs=pl.BlockSpec((_BS,), lambda i: (i,)),
        out_shape=jax.ShapeDtypeStruct((_T,), jnp.int32),
    )(b16, input)

    return lax.dynamic_update_slice(tc_full, sc_out, (_S_OFF,))


# speedup vs baseline: 1.0375x; 1.0375x over previous
import jax
import jax.numpy as jnp
from jax import lax
from jax.experimental import pallas as pl
from jax.experimental.pallas import tpu as pltpu
from jax.experimental.pallas import tpu_sc as plsc

_NUM_CLASSES = 10
_MAX_STEPS = 10
_T = 8388608
_NUM_STEPS = int(min(_MAX_STEPS, _T // 2))

_BIDX = (2022204, 2302723, 6800451, 5485289, 6417134,
         1160667, 5296668, 341701, 1583860)
_CM = (5, 2, 4, 2, 5, 4, 1, 8, 6, 5)

_NC, _NS, _L = 2, 16, 16
_NW = _NC * _NS
_SC_N = 2097152
_S_OFF = _T - _SC_N
_PW = _SC_N // _NW
_CH = 16384
_NCHUNK = _PW // _CH
_NPAIR = _NCHUNK // 2

_BS = 1572864
_TC_GRID = _S_OFF // _BS


def _sc_body(x_hbm, b_hbm, out_hbm,
             b_v, xb0, xb1, ob0, ob1, si0, si1, so0, so1):
    xbufs, obufs = (xb0, xb1), (ob0, ob1)
    sin, sout = (si0, si1), (so0, so1)
    wid = lax.axis_index("s") * _NC + lax.axis_index("c")
    base = _S_OFF + wid * _PW

    for b in range(2):
        pltpu.async_copy(x_hbm.at[pl.ds(base + b * _CH, _CH)], xbufs[b],
                         sin[b])

    pltpu.sync_copy(b_hbm, b_v)
    bvec = b_v[...]
    bs = [jnp.full((_L,), bvec[j], jnp.float32) for j in range(_NUM_STEPS - 1)]
    cms = [jnp.full((_L,), _CM[j], jnp.int32) for j in range(_NUM_STEPS)]

    def pair_body(p, _):
        for b in range(2):
            off = base + (2 * p + b) * _CH
            pltpu.make_async_copy(
                x_hbm.at[pl.ds(off, _CH)], xbufs[b], sin[b]).wait()

            @pl.when(p > 0)
            def _wait_out():
                pltpu.make_async_copy(
                    obufs[b], out_hbm.at[pl.ds(off - 2 * _CH - _S_OFF, _CH)],
                    sout[b]).wait()

            @plsc.parallel_loop(0, _CH, step=_L, unroll=8)
            def _vecs(i, b=b):
                x = xbufs[b][pl.ds(i, _L)]
                acc = cms[0]
                for j in range(_NUM_STEPS - 1):
                    acc = jnp.where(x > bs[j], cms[j + 1], acc)
                obufs[b][pl.ds(i, _L)] = acc

            pltpu.async_copy(obufs[b], out_hbm.at[pl.ds(off - _S_OFF, _CH)],
                             sout[b])

            @pl.when(p < _NPAIR - 1)
            def _issue_next():
                pltpu.async_copy(
                    x_hbm.at[pl.ds(off + 2 * _CH, _CH)], xbufs[b], sin[b])
        return 0

    lax.fori_loop(0, _NPAIR, pair_body, 0)
    for b in range(2):
        pltpu.make_async_copy(
            obufs[b],
            out_hbm.at[pl.ds(wid * _PW + (_NCHUNK - 2 + b) * _CH, _CH)],
            sout[b]).wait()


def _tc_body(b_s, x_ref, out_ref):
    x = x_ref[...]
    acc = jnp.full(x.shape, _CM[0], jnp.int32)
    for j in range(_NUM_STEPS - 1):
        acc = jnp.where(x > b_s[j], jnp.int32(_CM[j + 1]), acc)
    out_ref[...] = acc


def kernel(input):
    boundaries = jnp.sort(input[jnp.array(_BIDX, jnp.int32)])
    b16 = jnp.zeros((_L,), jnp.float32).at[: _NUM_STEPS - 1].set(boundaries)

    mesh = plsc.VectorSubcoreMesh(core_axis_name="c", subcore_axis_name="s")
    sc_run = pl.kernel(
        _sc_body,
        out_type=jax.ShapeDtypeStruct((_SC_N,), jnp.int32),
        mesh=mesh,
        scratch_types=[
            pltpu.VMEM((_L,), jnp.float32),
            pltpu.VMEM((_CH,), jnp.float32),
            pltpu.VMEM((_CH,), jnp.float32),
            pltpu.VMEM((_CH,), jnp.int32),
            pltpu.VMEM((_CH,), jnp.int32),
            pltpu.SemaphoreType.DMA,
            pltpu.SemaphoreType.DMA,
            pltpu.SemaphoreType.DMA,
            pltpu.SemaphoreType.DMA,
        ],
    )
    sc_out = sc_run(input, b16)

    tc_full = pl.pallas_call(
        _tc_body,
        grid=(_TC_GRID,),
        in_specs=[
            pl.BlockSpec(memory_space=pltpu.SMEM),
            pl.BlockSpec((_BS,), lambda i: (i,)),
        ],
        out_specs=pl.BlockSpec((_BS,), lambda i: (i,)),
        out_shape=jax.ShapeDtypeStruct((_T,), jnp.int32),
    )(b16, input)

    return lax.dynamic_update_slice(tc_full, sc_out, (_S_OFF,))


# speedup vs baseline: 1.0775x; 1.0385x over previous
import jax
import jax.numpy as jnp
from jax import lax
from jax.experimental import pallas as pl
from jax.experimental.pallas import tpu as pltpu
from jax.experimental.pallas import tpu_sc as plsc

_NUM_CLASSES = 10
_MAX_STEPS = 10
_T = 8388608
_NUM_STEPS = int(min(_MAX_STEPS, _T // 2))

_BIDX = (2022204, 2302723, 6800451, 5485289, 6417134,
         1160667, 5296668, 341701, 1583860)
_CM = (5, 2, 4, 2, 5, 4, 1, 8, 6, 5)

_NC, _NS, _L = 2, 16, 16
_NW = _NC * _NS
_SC_N = 2097152
_S_OFF = _T - _SC_N
_PW = _SC_N // _NW
_CH = 16384
_NCHUNK = _PW // _CH
_NPAIR = _NCHUNK // 2

_BS = 1048576
_TC_GRID = _S_OFF // _BS


def _sc_body(x_hbm, b_hbm, out_hbm,
             b_v, xb0, xb1, ob0, ob1, si0, si1, so0, so1):
    xbufs, obufs = (xb0, xb1), (ob0, ob1)
    sin, sout = (si0, si1), (so0, so1)
    wid = lax.axis_index("s") * _NC + lax.axis_index("c")
    base = _S_OFF + wid * _PW

    for b in range(2):
        pltpu.async_copy(x_hbm.at[pl.ds(base + b * _CH, _CH)], xbufs[b],
                         sin[b])

    pltpu.sync_copy(b_hbm, b_v)
    bvec = b_v[...]
    bs = [jnp.full((_L,), bvec[j], jnp.float32) for j in range(_NUM_STEPS - 1)]
    cms = [jnp.full((_L,), _CM[j], jnp.int32) for j in range(_NUM_STEPS)]

    def pair_body(p, _):
        for b in range(2):
            off = base + (2 * p + b) * _CH
            pltpu.make_async_copy(
                x_hbm.at[pl.ds(off, _CH)], xbufs[b], sin[b]).wait()

            @pl.when(p > 0)
            def _wait_out():
                pltpu.make_async_copy(
                    obufs[b], out_hbm.at[pl.ds(off - 2 * _CH - _S_OFF, _CH)],
                    sout[b]).wait()

            @plsc.parallel_loop(0, _CH, step=_L, unroll=8)
            def _vecs(i, b=b):
                x = xbufs[b][pl.ds(i, _L)]
                acc = cms[0]
                for j in range(_NUM_STEPS - 1):
                    acc = jnp.where(x > bs[j], cms[j + 1], acc)
                obufs[b][pl.ds(i, _L)] = acc

            pltpu.async_copy(obufs[b], out_hbm.at[pl.ds(off - _S_OFF, _CH)],
                             sout[b])

            @pl.when(p < _NPAIR - 1)
            def _issue_next():
                pltpu.async_copy(
                    x_hbm.at[pl.ds(off + 2 * _CH, _CH)], xbufs[b], sin[b])
        return 0

    lax.fori_loop(0, _NPAIR, pair_body, 0)
    for b in range(2):
        pltpu.make_async_copy(
            obufs[b],
            out_hbm.at[pl.ds(wid * _PW + (_NCHUNK - 2 + b) * _CH, _CH)],
            sout[b]).wait()


def _tc_body(b_s, x_ref, out_ref):
    x = x_ref[...]
    acc = jnp.full(x.shape, _CM[0], jnp.int32)
    for j in range(_NUM_STEPS - 1):
        acc = jnp.where(x > b_s[j], jnp.int32(_CM[j + 1]), acc)
    out_ref[...] = acc


def kernel(input):
    boundaries = jnp.sort(input[jnp.array(_BIDX, jnp.int32)])
    b16 = jnp.zeros((_L,), jnp.float32).at[: _NUM_STEPS - 1].set(boundaries)

    mesh = plsc.VectorSubcoreMesh(core_axis_name="c", subcore_axis_name="s")
    sc_run = pl.kernel(
        _sc_body,
        out_type=jax.ShapeDtypeStruct((_SC_N,), jnp.int32),
        mesh=mesh,
        scratch_types=[
            pltpu.VMEM((_L,), jnp.float32),
            pltpu.VMEM((_CH,), jnp.float32),
            pltpu.VMEM((_CH,), jnp.float32),
            pltpu.VMEM((_CH,), jnp.int32),
            pltpu.VMEM((_CH,), jnp.int32),
            pltpu.SemaphoreType.DMA,
            pltpu.SemaphoreType.DMA,
            pltpu.SemaphoreType.DMA,
            pltpu.SemaphoreType.DMA,
        ],
    )
    sc_out = sc_run(input, b16)

    tc_full = pl.pallas_call(
        _tc_body,
        grid=(_TC_GRID,),
        in_specs=[
            pl.BlockSpec(memory_space=pltpu.SMEM),
            pl.BlockSpec((_BS,), lambda i: (i,)),
        ],
        out_specs=pl.BlockSpec((_BS,), lambda i: (i,)),
        out_shape=jax.ShapeDtypeStruct((_T,), jnp.int32),
    )(b16, input)

    return lax.dynamic_update_slice(tc_full, sc_out, (_S_OFF,))
